# baseline (speedup 1.0000x reference)
import jax
import jax.numpy as jnp
from jax import lax
from jax.experimental import pallas as pl
from jax.experimental.pallas import tpu as pltpu
from jax.experimental.pallas import tpu_sc as plsc

N_ROWS = 1_000_000
D = 32
N_IDX = 16_384
WCOLS = 3584
MAIN_COLS = 999_936
N_WIN = MAIN_COLS // WCOLS
N_WORKERS = 32
VCH = 56
SVB = 64

_mesh = plsc.VectorSubcoreMesh(core_axis_name="c", subcore_axis_name="s")


def _sc_body(xt_hbm, sidx_hbm, sval_hbm, order_hbm, wstart_hbm, out_hbm,
             win_v, sidx_c, sval_v, order_c, wstart_v):
  nc = 2
  wid = lax.axis_index("s") * nc + lax.axis_index("c")

  pltpu.sync_copy(wstart_hbm, wstart_v.at[pl.ds(0, N_WIN + 5)])

  iota16 = lax.iota(jnp.int32, 16)

  @pl.loop(wid, N_WIN, step=N_WORKERS)
  def _window(w):
    col0 = w * WCOLS
    pltpu.sync_copy(xt_hbm.at[:, pl.ds(col0, WCOLS)], win_v)
    bounds = plsc.load_gather(wstart_v, [w + iota16])
    s0 = bounds[0]
    e0 = bounds[1]
    nch = (e0 - s0 + (VCH - 1)) // VCH

    @pl.loop(0, nch)
    def _chunk(k):
      off = s0 + k * VCH
      offc = jnp.minimum((off // 8) * 8, N_IDX - SVB)
      pltpu.sync_copy(sidx_hbm.at[pl.ds(offc, SVB)], sidx_c)
      pltpu.sync_copy(order_hbm.at[pl.ds(offc, SVB)], order_c)
      pltpu.sync_copy(sval_hbm.at[order_c], sval_v)
      lim = jnp.minimum(e0 - off, VCH)

      @pl.loop(0, lim)
      def _pos(t):
        r = off + t - offc
        rsplat = jnp.full((16,), r, jnp.int32)
        cvec = plsc.load_gather(sidx_c, [rsplat])
        cvec = cvec - col0
        v0 = plsc.load_gather(sval_v, [rsplat, iota16])
        v1 = plsc.load_gather(sval_v, [rsplat, iota16 + 16])
        plsc.addupdate_scatter(win_v, [iota16, cvec], v0)
        plsc.addupdate_scatter(win_v, [iota16 + 16, cvec], v1)

    pltpu.sync_copy(win_v, out_hbm.at[:, pl.ds(col0, WCOLS)])


_sc_sweep = pl.kernel(
    _sc_body,
    out_type=jax.ShapeDtypeStruct((D, N_ROWS), jnp.float32),
    mesh=_mesh,
    scratch_types=[
        pltpu.VMEM((D, WCOLS), jnp.float32),
        pltpu.VMEM((SVB,), jnp.int32),
        pltpu.VMEM((SVB, 128), jnp.float32),
        pltpu.VMEM((SVB,), jnp.int32),
        pltpu.VMEM((N_WIN + 21,), jnp.int32),
    ],
    compiler_params=pltpu.CompilerParams(needs_layout_passes=False),
)


def kernel(x, indices, values):
  idx = indices.astype(jnp.int32)
  sidx, order = lax.sort(
      (idx, jnp.arange(N_IDX, dtype=jnp.int32)), num_keys=1,
      is_stable=False)
  svals = jnp.pad(values.astype(jnp.float32), ((0, 0), (0, 96)))
  wstart = jnp.searchsorted(
      sidx, jnp.arange(N_WIN + 5, dtype=jnp.int32) * WCOLS,
      side="left", method="compare_all").astype(jnp.int32)

  out_t = _sc_sweep(x.T, sidx, svals, order, wstart)

  tail_rows = jnp.arange(N_ROWS - MAIN_COLS, dtype=jnp.int32) + MAIN_COLS
  onehot = (tail_rows[:, None] == idx[None, :]).astype(jnp.float32)
  tail_out = x[MAIN_COLS:, :] + jnp.matmul(onehot, values, precision=lax.Precision.HIGHEST)
  out = out_t.T
  return lax.dynamic_update_slice(out, tail_out, (MAIN_COLS, 0))


# speedup vs baseline: 1.0452x; 1.0452x over previous
import jax
import jax.numpy as jnp
from jax import lax
from jax.experimental import pallas as pl
from jax.experimental.pallas import tpu as pltpu
from jax.experimental.pallas import tpu_sc as plsc

N_ROWS = 1_000_000
D = 32
N_IDX = 16_384
WCOLS = 2688
MAIN_COLS = 999_936
N_WIN = MAIN_COLS // WCOLS
N_WORKERS = 32
VCH = 56
SVB = 64

_mesh = plsc.VectorSubcoreMesh(core_axis_name="c", subcore_axis_name="s")


def _sc_body(xt_hbm, sidx_hbm, sval_hbm, order_hbm, wstart_hbm, out_hbm,
             win_v, sidx_v, sval_v, order_v, wstart_v):
  nc = 2
  wid = lax.axis_index("s") * nc + lax.axis_index("c")

  pltpu.sync_copy(sidx_hbm, sidx_v.at[pl.ds(0, N_IDX)])
  pltpu.sync_copy(order_hbm, order_v.at[pl.ds(0, N_IDX)])
  pltpu.sync_copy(wstart_hbm, wstart_v.at[pl.ds(0, N_WIN + 5)])

  iota16 = lax.iota(jnp.int32, 16)

  @pl.loop(wid, N_WIN, step=N_WORKERS)
  def _window(w):
    col0 = w * WCOLS
    pltpu.sync_copy(xt_hbm.at[:, pl.ds(col0, WCOLS)], win_v)
    bounds = plsc.load_gather(wstart_v, [w + iota16])
    s0 = bounds[0]
    e0 = bounds[1]
    nch = (e0 - s0 + (VCH - 1)) // VCH

    @pl.loop(0, nch)
    def _chunk(k):
      off = s0 + k * VCH
      offc = jnp.minimum((off // 8) * 8, N_IDX - SVB)
      pltpu.sync_copy(sval_hbm.at[order_v.at[pl.ds(offc, SVB)]], sval_v)
      lim = jnp.minimum(e0 - off, VCH)

      @pl.loop(0, lim)
      def _pos(t):
        p = off + t
        rsplat = jnp.full((16,), p - offc, jnp.int32)
        cvec = plsc.load_gather(sidx_v, [jnp.full((16,), p, jnp.int32)])
        cvec = cvec - col0
        v0 = plsc.load_gather(sval_v, [rsplat, iota16])
        v1 = plsc.load_gather(sval_v, [rsplat, iota16 + 16])
        plsc.addupdate_scatter(win_v, [iota16, cvec], v0)
        plsc.addupdate_scatter(win_v, [iota16 + 16, cvec], v1)

    pltpu.sync_copy(win_v, out_hbm.at[:, pl.ds(col0, WCOLS)])


_sc_sweep = pl.kernel(
    _sc_body,
    out_type=jax.ShapeDtypeStruct((D, N_ROWS), jnp.float32),
    mesh=_mesh,
    scratch_types=[
        pltpu.VMEM((D, WCOLS), jnp.float32),
        pltpu.VMEM((N_IDX + 16,), jnp.int32),
        pltpu.VMEM((SVB, 128), jnp.float32),
        pltpu.VMEM((N_IDX + 16,), jnp.int32),
        pltpu.VMEM((N_WIN + 21,), jnp.int32),
    ],
    compiler_params=pltpu.CompilerParams(needs_layout_passes=False),
)


def kernel(x, indices, values):
  idx = indices.astype(jnp.int32)
  sidx, order = lax.sort(
      (idx, jnp.arange(N_IDX, dtype=jnp.int32)), num_keys=1,
      is_stable=False)
  svals = jnp.pad(values.astype(jnp.float32), ((0, 0), (0, 96)))
  wstart = jnp.searchsorted(
      sidx, jnp.arange(N_WIN + 5, dtype=jnp.int32) * WCOLS,
      side="left", method="compare_all").astype(jnp.int32)

  out_t = _sc_sweep(x.T, sidx, svals, order, wstart)

  tail_rows = jnp.arange(N_ROWS - MAIN_COLS, dtype=jnp.int32) + MAIN_COLS
  onehot = (tail_rows[:, None] == idx[None, :]).astype(jnp.float32)
  tail_out = x[MAIN_COLS:, :] + jnp.matmul(onehot, values, precision=lax.Precision.HIGHEST)
  out = out_t.T
  return lax.dynamic_update_slice(out, tail_out, (MAIN_COLS, 0))
